# baseline (speedup 1.0000x reference)
import functools

import jax
import jax.numpy as jnp
from jax import lax
from jax.experimental import pallas as pl
from jax.experimental.pallas import tpu as pltpu
from jax.experimental.pallas import tpu_sc as plsc

NC = 2
NS = 16
NW = NC * NS
D = 64
K = 128


@functools.partial(jax.jit, static_argnames=())
def _gather(idx3, table):
    n_chunks = idx3.shape[1]
    b_per_w = n_chunks * K
    n = NW * b_per_w
    mesh = plsc.VectorSubcoreMesh(core_axis_name="c", subcore_axis_name="s")

    @functools.partial(
        pl.kernel,
        out_type=jax.ShapeDtypeStruct((n, D), jnp.float32),
        mesh=mesh,
        scratch_types=[
            pltpu.VMEM((n_chunks, K), jnp.int32),
            pltpu.VMEM((K, D), jnp.float32),
            pltpu.SemaphoreType.DMA,
            pltpu.SemaphoreType.DMA,
        ],
        compiler_params=pltpu.CompilerParams(use_tc_tiling_on_sc=False),
    )
    def k(idx_hbm, table_hbm, out_hbm, idx_v, rows_v, sem_i, sem_g):
        wid = lax.axis_index("s") * NC + lax.axis_index("c")
        base = wid * b_per_w
        pltpu.async_copy(idx_hbm.at[wid], idx_v, sem_i).wait()

        def chunk(j, carry):
            pltpu.async_copy(table_hbm.at[idx_v.at[j]], rows_v, sem_g).wait()
            pltpu.sync_copy(rows_v, out_hbm.at[pl.ds(base + j * K, K)])
            return carry

        lax.fori_loop(0, n_chunks, chunk, 0)

    return k(idx3, table)


def kernel(token_seq, table):
    b, s = token_seq.shape
    n = b * s
    idx3 = token_seq.reshape(NW, n // (NW * K), K)
    out = _gather(idx3, table)
    return out.reshape(b, s, D)


# speedup vs baseline: 1.1148x; 1.1148x over previous
import functools

import jax
import jax.numpy as jnp
from jax import lax
from jax.experimental import pallas as pl
from jax.experimental.pallas import tpu as pltpu
from jax.experimental.pallas import tpu_sc as plsc

NC = 2
NS = 16
NW = NC * NS
D = 64
K = 128
NBUF = 4


def _gather(idx3, table):
    n_chunks = idx3.shape[1]
    b_per_w = n_chunks * K
    n = NW * b_per_w
    mesh = plsc.VectorSubcoreMesh(core_axis_name="c", subcore_axis_name="s")

    @functools.partial(
        pl.kernel,
        out_type=jax.ShapeDtypeStruct((n, D), jnp.float32),
        mesh=mesh,
        scratch_types=[
            pltpu.VMEM((n_chunks, K), jnp.int32),
            pltpu.VMEM((NBUF, K, D), jnp.float32),
            pltpu.SemaphoreType.DMA,
            pltpu.SemaphoreType.DMA,
            pltpu.SemaphoreType.DMA,
        ],
        compiler_params=pltpu.CompilerParams(use_tc_tiling_on_sc=False),
    )
    def k(idx_hbm, table_hbm, out_hbm, idx_v, rows_v, sem_i, sem_g, sem_o):
        wid = lax.axis_index("s") * NC + lax.axis_index("c")
        base = wid * b_per_w
        pltpu.async_copy(idx_hbm.at[wid], idx_v, sem_i).wait()

        def start_gather(m, buf):
            pltpu.async_copy(table_hbm.at[idx_v.at[m]], rows_v.at[buf], sem_g)

        def start_out(j, buf):
            pltpu.async_copy(
                rows_v.at[buf], out_hbm.at[pl.ds(base + j * K, K)], sem_o
            )

        def wait_gather():
            pltpu.make_async_copy(
                table_hbm.at[idx_v.at[0]], rows_v.at[0], sem_g
            ).wait()

        def wait_out():
            pltpu.make_async_copy(
                rows_v.at[0], out_hbm.at[pl.ds(base, K)], sem_o
            ).wait()

        for b in range(NBUF - 1):
            start_gather(b, b)
        start_gather(NBUF - 1, NBUF - 1)
        wait_gather()
        start_out(0, 0)

        n_main = n_chunks - NBUF
        assert n_main % NBUF == 0

        def group(g, carry):
            for b in range(NBUF):
                j = 1 + g * NBUF + b
                wait_out()
                start_gather(j + NBUF - 1, b)
                wait_gather()
                start_out(j, (b + 1) % NBUF)
            return carry

        lax.fori_loop(0, n_main // NBUF, group, 0)

        for j in range(n_chunks - NBUF + 1, n_chunks):
            wait_out()
            wait_gather()
            start_out(j, j % NBUF)
        wait_out()

    return k(idx3, table)


def kernel(token_seq, table):
    b, s = token_seq.shape
    n = b * s
    idx3 = token_seq.reshape(NW, n // (NW * K), K)
    out = _gather(idx3, table)
    return out.reshape(b, s, D)
